# Initial kernel scaffold; baseline (speedup 1.0000x reference)
import jax
import jax.numpy as jnp
from jax.experimental import pallas as pl


def kernel(x, edge_index, batch, W1, b1, W2, b2, W3, b3, Wl, bl):
    raise NotImplementedError("write your pallas kernel here")


# speedup vs baseline: 12.0457x; 12.0457x over previous
import functools

import jax
import jax.numpy as jnp
from jax import lax
from jax.experimental import pallas as pl
from jax.experimental.pallas import tpu as pltpu
from jax.experimental.pallas import tpu_sc as plsc

N = 100000
E = 1600000
G = 512
EROWS = E // 128
PADROWS = 12800
NSC = 2
NTILE = 16
C = 11200
NCHUNK = 9
ROUNDS = 5
B_FIRE = 1024
B_CAP = B_FIRE + 128
JMAX = B_FIRE // 128

_mesh = functools.partial(
    plsc.VectorSubcoreMesh, core_axis_name="c", subcore_axis_name="s")


def _zero_vmem(ref, n):
  z = jnp.zeros((16,), ref.dtype)
  def body(j, _):
    off = jnp.minimum(j * 16, n - 16)
    ref[pl.ds(off, 16)] = z
    return 0
  lax.fori_loop(0, (n + 15) // 16, body, 0)


def _deg_kernel(dst2d):
  @functools.partial(
      pl.kernel,
      out_type=jax.ShapeDtypeStruct((NSC, N), jnp.float32),
      mesh=_mesh(),
      scratch_types=[
          pltpu.VMEM((16, 128), jnp.int32),
          pltpu.VMEM((128,), jnp.float32),
          pltpu.VMEM((6256,), jnp.float32),
          pltpu.VMEM_SHARED((N,), jnp.float32),
      ],
      compiler_params=pltpu.CompilerParams(needs_layout_passes=False),
  )
  def k(dst_hbm, out_hbm, dstw, ones_v, zbuf, deg_sp):
    c = lax.axis_index("c")
    s = lax.axis_index("s")
    w32 = s * NSC + c

    one = jnp.ones((16,), jnp.float32)
    for j in range(8):
      ones_v[pl.ds(j * 16, 16)] = one
    _zero_vmem(zbuf, 6256)
    zst = jnp.minimum(s * 6256, N - 6256)
    pltpu.sync_copy(zbuf, deg_sp.at[pl.ds(zst, 6256)])
    plsc.subcore_barrier()

    def win(w, _):
      base = w32 * 400 + w * 16
      pltpu.sync_copy(dst_hbm.at[pl.ds(base, 16)], dstw)
      def row(r, _):
        @pl.when(base + r < EROWS)
        def _():
          pltpu.sync_copy(ones_v, deg_sp.at[dstw.at[r]], add=True)
        return 0
      lax.fori_loop(0, 16, row, 0)
      return 0
    lax.fori_loop(0, 25, win, 0)

    plsc.subcore_barrier()
    @pl.when(s == 0)
    def _():
      pltpu.sync_copy(deg_sp, out_hbm.at[c])

  return k(dst2d)


def _make_agg(F):
  @functools.partial(
      pl.kernel,
      out_type=jax.ShapeDtypeStruct((N, F), jnp.float32),
      mesh=_mesh(),
      scratch_types=[
          pltpu.VMEM((16, 128), jnp.int32),
          pltpu.VMEM((16, 128), jnp.int32),
          pltpu.VMEM((B_CAP,), jnp.int32),
          pltpu.VMEM((B_CAP,), jnp.int32),
          pltpu.VMEM((JMAX, 128), jnp.int32),
          pltpu.VMEM((JMAX, 128), jnp.int32),
          pltpu.VMEM((2, 128, F), jnp.float32),
          pltpu.VMEM_SHARED((C + 16, F), jnp.float32),
          pltpu.SemaphoreType.DMA,
          pltpu.SemaphoreType.DMA,
      ],
      compiler_params=pltpu.CompilerParams(needs_layout_passes=False,
                                           use_tc_tiling_on_sc=False),
  )
  def k(t_hbm, src_hbm, dst_hbm, out_hbm,
        srcw, dstw, sapp, dapp, sfire, dfire, rows, acc_sp, gsem, ssem):
    cc = lax.axis_index("c")
    s = lax.axis_index("s")
    lane = lax.iota(jnp.int32, 16)

    def copy_fire(j):
      for jj in range(8):
        sfire[j, pl.ds(jj * 16, 16)] = sapp[pl.ds(j * 128 + jj * 16, 16)]
        dfire[j, pl.ds(jj * 16, 16)] = dapp[pl.ds(j * 128 + jj * 16, 16)]

    def fire():
      for j in range(JMAX):
        copy_fire(j)
      def gth(j, b):
        return pltpu.make_async_copy(t_hbm.at[sfire.at[j]], rows.at[b], gsem)
      def sct(j, b):
        return pltpu.make_async_copy(rows.at[b], acc_sp.at[dfire.at[j]], ssem)
      gth(0, 0).start()
      for j in range(JMAX):
        b = j % 2
        gth(j, b).wait()
        if j + 1 < JMAX:
          if j >= 1:
            sct(j - 1, (j - 1) % 2).wait()
          gth(j + 1, (j + 1) % 2).start()
        sct(j, b).start(add=True)
      sct(JMAX - 2, JMAX % 2).wait()
      sct(JMAX - 1, (JMAX - 1) % 2).wait()

    def rounds(rr, _):
      ck = rr * NSC + cc

      @pl.when(ck < NCHUNK)
      def _round():
        lo = ck * C
        limit = jnp.minimum(lo + C, N) - lo
        st = jnp.minimum(s * 800, limit - 800)
        pltpu.sync_copy(t_hbm.at[pl.ds(lo + st, 800)], acc_sp.at[pl.ds(st, 800)])
        plsc.subcore_barrier()

        def win(w, cur):
          base = s * 784 + w * 16
          pltpu.sync_copy(src_hbm.at[pl.ds(base, 16)], srcw)
          pltpu.sync_copy(dst_hbm.at[pl.ds(base, 16)], dstw)

          def row(r, cur):
            valid = (base + r) < EROWS
            for jj in range(8):
              dv = dstw[r, pl.ds(jj * 16, 16)]
              sv = srcw[r, pl.ds(jj * 16, 16)]
              m = jnp.logical_and(dv >= lo, dv < lo + C)
              m = jnp.logical_and(m, valid)
              plsc.store_compressed(dapp.at[pl.ds(cur, 16)], dv - lo, mask=m)
              plsc.store_compressed(sapp.at[pl.ds(cur, 16)], sv, mask=m)
              cur = cur + plsc.all_reduce_population_count(m)[0]

            def do_fire(cur):
              fire()
              for jj in range(8):
                sapp[pl.ds(jj * 16, 16)] = sapp[pl.ds(B_FIRE + jj * 16, 16)]
                dapp[pl.ds(jj * 16, 16)] = dapp[pl.ds(B_FIRE + jj * 16, 16)]
              return cur - B_FIRE
            return lax.cond(cur >= B_FIRE, do_fire, lambda cur: cur, cur)

          return lax.fori_loop(0, 16, row, cur)

        cur = lax.fori_loop(0, 49, win, jnp.int32(0))

        def pad(jj, _):
          idx = jj * 16 + lane
          keep = idx < cur
          dsrc = jnp.remainder(idx * 797 + s * 4099, N)
          sapp[pl.ds(jj * 16, 16)] = jnp.where(keep, sapp[pl.ds(jj * 16, 16)],
                                               dsrc)
          dapp[pl.ds(jj * 16, 16)] = jnp.where(keep, dapp[pl.ds(jj * 16, 16)],
                                               C + lane)
          return 0
        lax.fori_loop(0, B_CAP // 16, pad, 0)
        fire()

        plsc.subcore_barrier()
        pltpu.sync_copy(acc_sp.at[pl.ds(st, 800)],
                        out_hbm.at[pl.ds(lo + st, 800)])
        plsc.subcore_barrier()
      return 0

    lax.fori_loop(0, ROUNDS, rounds, 0)

  return k


_agg64 = _make_agg(64)
_agg128 = _make_agg(128)


BLK = 2000
NBLK = N // BLK


def _dinv_kernel(deg_partT):
  def body(deg_ref, out_ref):
    d = deg_ref[:, 0:1] + deg_ref[:, 1:2] + 1.0
    out_ref[...] = lax.rsqrt(d)
  return pl.pallas_call(
      body,
      grid=(NBLK,),
      in_specs=[pl.BlockSpec((BLK, NSC), lambda i: (i, 0))],
      out_specs=pl.BlockSpec((BLK, 1), lambda i: (i, 0)),
      out_shape=jax.ShapeDtypeStruct((N, 1), jnp.float32),
  )(deg_partT)


def _mm(h, Wm, dinv2, bias2, relu_pre):
  Fin, Fout = Wm.shape
  def body(h_ref, w_ref, dinv_ref, b_ref, out_ref):
    hb = h_ref[...]
    dv = dinv_ref[...]
    if relu_pre:
      hb = jnp.maximum(hb * dv + b_ref[...], 0.0)
    t = jnp.dot(hb, w_ref[...], preferred_element_type=jnp.float32)
    out_ref[...] = t * dv
  return pl.pallas_call(
      body,
      grid=(NBLK,),
      in_specs=[
          pl.BlockSpec((BLK, Fin), lambda i: (i, 0)),
          pl.BlockSpec((Fin, Fout), lambda i: (0, 0)),
          pl.BlockSpec((BLK, 1), lambda i: (i, 0)),
          pl.BlockSpec((1, Fin), lambda i: (0, 0)),
      ],
      out_specs=pl.BlockSpec((BLK, Fout), lambda i: (i, 0)),
      out_shape=jax.ShapeDtypeStruct((N, Fout), jnp.float32),
  )(h, Wm, dinv2, bias2)


def _pool_head(agg3, dinv2, b32, batch2d, Wl, bl2):
  def body(a_ref, dinv_ref, b3_ref, bt_ref, wl_ref, bl_ref, out_ref,
           s_acc, cnt_acc):
    i = pl.program_id(0)
    @pl.when(i == 0)
    def _():
      s_acc[...] = jnp.zeros_like(s_acc)
      cnt_acc[...] = jnp.zeros_like(cnt_acc)
    h = a_ref[...] * dinv_ref[...] + b3_ref[...]
    gids = lax.broadcasted_iota(jnp.int32, (BLK, G), 1)
    oh = (gids == bt_ref[...]).astype(jnp.float32)
    dnum = (((0,), (0,)), ((), ()))
    s_acc[...] += lax.dot_general(oh, h, dimension_numbers=dnum,
                                  preferred_element_type=jnp.float32)
    cnt_acc[...] += lax.dot_general(oh, jnp.ones((BLK, 1), jnp.float32),
                                    dimension_numbers=dnum,
                                    preferred_element_type=jnp.float32)
    @pl.when(i == NBLK - 1)
    def _():
      pooled = s_acc[...] / jnp.maximum(cnt_acc[...], 1.0)
      logits = jnp.dot(pooled, wl_ref[...],
                       preferred_element_type=jnp.float32) + bl_ref[...]
      mx = jnp.max(logits, axis=-1, keepdims=True)
      lse = mx + jnp.log(jnp.sum(jnp.exp(logits - mx), axis=-1, keepdims=True))
      out_ref[...] = logits - lse
  return pl.pallas_call(
      body,
      grid=(NBLK,),
      in_specs=[
          pl.BlockSpec((BLK, 128), lambda i: (i, 0)),
          pl.BlockSpec((BLK, 1), lambda i: (i, 0)),
          pl.BlockSpec((1, 128), lambda i: (0, 0)),
          pl.BlockSpec((BLK, 1), lambda i: (i, 0)),
          pl.BlockSpec((128, 14), lambda i: (0, 0)),
          pl.BlockSpec((1, 14), lambda i: (0, 0)),
      ],
      out_specs=pl.BlockSpec((G, 14), lambda i: (0, 0)),
      out_shape=jax.ShapeDtypeStruct((G, 14), jnp.float32),
      scratch_shapes=[
          pltpu.VMEM((G, 128), jnp.float32),
          pltpu.VMEM((G, 1), jnp.float32),
      ],
  )(agg3, dinv2, b32, batch2d, Wl, bl2)


def kernel(x, edge_index, batch, W1, b1, W2, b2, W3, b3, Wl, bl):
  pad = PADROWS * 128 - E
  src2d = jnp.concatenate(
      [edge_index[0], jnp.zeros((pad,), jnp.int32)]).reshape(PADROWS, 128)
  dst2d = jnp.concatenate(
      [edge_index[1], jnp.zeros((pad,), jnp.int32)]).reshape(PADROWS, 128)

  deg_part = _deg_kernel(dst2d)
  dinv2 = _dinv_kernel(deg_part.T)

  zero20 = jnp.zeros((1, 20), jnp.float32)
  t1 = _mm(x, W1, dinv2, zero20, relu_pre=False)
  a1 = _agg64(t1, src2d, dst2d)
  t2 = _mm(a1, W2, dinv2, b1.reshape(1, -1), relu_pre=True)
  a2 = _agg64(t2, src2d, dst2d)
  t3 = _mm(a2, W3, dinv2, b2.reshape(1, -1), relu_pre=True)
  a3 = _agg128(t3, src2d, dst2d)

  return _pool_head(a3, dinv2, b3.reshape(1, -1), batch.reshape(N, 1), Wl,
                    bl.reshape(1, -1))


# speedup vs baseline: 17.9509x; 1.4902x over previous
import functools

import jax
import jax.numpy as jnp
from jax import lax
from jax.experimental import pallas as pl
from jax.experimental.pallas import tpu as pltpu
from jax.experimental.pallas import tpu_sc as plsc

N = 100000
E = 1600000
G = 512
EROWS = E // 128
PADROWS = 12800
NSC = 2
NTILE = 16
C = 11200
NCHUNK = 9
ROUNDS = 5
B_FIRE = 1024
B_CAP = B_FIRE + 128
JMAX = B_FIRE // 128

_mesh = functools.partial(
    plsc.VectorSubcoreMesh, core_axis_name="c", subcore_axis_name="s")


def _zero_vmem(ref, n):
  z = jnp.zeros((16,), ref.dtype)
  def body(j, _):
    off = jnp.minimum(j * 16, n - 16)
    ref[pl.ds(off, 16)] = z
    return 0
  lax.fori_loop(0, (n + 15) // 16, body, 0)


def _deg_kernel(dst2d):
  @functools.partial(
      pl.kernel,
      out_type=jax.ShapeDtypeStruct((NSC, N), jnp.float32),
      mesh=_mesh(),
      scratch_types=[
          pltpu.VMEM((16, 128), jnp.int32),
          pltpu.VMEM((128,), jnp.float32),
          pltpu.VMEM((6256,), jnp.float32),
          pltpu.VMEM_SHARED((N,), jnp.float32),
      ],
      compiler_params=pltpu.CompilerParams(needs_layout_passes=False),
  )
  def k(dst_hbm, out_hbm, dstw, ones_v, zbuf, deg_sp):
    c = lax.axis_index("c")
    s = lax.axis_index("s")
    w32 = s * NSC + c

    one = jnp.ones((16,), jnp.float32)
    for j in range(8):
      ones_v[pl.ds(j * 16, 16)] = one
    _zero_vmem(zbuf, 6256)
    zst = jnp.minimum(s * 6256, N - 6256)
    pltpu.sync_copy(zbuf, deg_sp.at[pl.ds(zst, 6256)])
    plsc.subcore_barrier()

    def win(w, _):
      base = w32 * 400 + w * 16
      pltpu.sync_copy(dst_hbm.at[pl.ds(base, 16)], dstw)
      def row(r, _):
        @pl.when(base + r < EROWS)
        def _():
          pltpu.sync_copy(ones_v, deg_sp.at[dstw.at[r]], add=True)
        return 0
      lax.fori_loop(0, 16, row, 0)
      return 0
    lax.fori_loop(0, 25, win, 0)

    plsc.subcore_barrier()
    @pl.when(s == 0)
    def _():
      pltpu.sync_copy(deg_sp, out_hbm.at[c])

  return k(dst2d)


def _make_agg(F, C, rounds_n):
  Q = ((C // 16 + 7) // 8) * 8
  @functools.partial(
      pl.kernel,
      out_type=jax.ShapeDtypeStruct((N, F), jnp.float32),
      mesh=_mesh(),
      scratch_types=[
          pltpu.VMEM((2, 16, 128), jnp.int32),
          pltpu.VMEM((2, 16, 128), jnp.int32),
          pltpu.VMEM((B_CAP,), jnp.int32),
          pltpu.VMEM((B_CAP,), jnp.int32),
          pltpu.VMEM((JMAX, 128), jnp.int32),
          pltpu.VMEM((JMAX, 128), jnp.int32),
          pltpu.VMEM((2, 128, F), jnp.float32),
          pltpu.VMEM_SHARED((C + 16, F), jnp.float32),
          pltpu.SemaphoreType.DMA,
          pltpu.SemaphoreType.DMA,
          pltpu.SemaphoreType.DMA,
      ],
      compiler_params=pltpu.CompilerParams(needs_layout_passes=False,
                                           use_tc_tiling_on_sc=False),
  )
  def k(t_hbm, src_hbm, dst_hbm, out_hbm,
        srcw, dstw, sapp, dapp, sfire, dfire, rows, acc_sp, gsem, ssem, wsem):
    cc = lax.axis_index("c")
    s = lax.axis_index("s")
    lane = lax.iota(jnp.int32, 16)

    def copy_fire(j):
      for jj in range(8):
        sfire[j, pl.ds(jj * 16, 16)] = sapp[pl.ds(j * 128 + jj * 16, 16)]
        dfire[j, pl.ds(jj * 16, 16)] = dapp[pl.ds(j * 128 + jj * 16, 16)]

    def fire():
      for j in range(JMAX):
        copy_fire(j)
      def gth(j, b):
        return pltpu.make_async_copy(t_hbm.at[sfire.at[j]], rows.at[b], gsem)
      def sct(j, b):
        return pltpu.make_async_copy(rows.at[b], acc_sp.at[dfire.at[j]], ssem)
      gth(0, 0).start()
      for j in range(JMAX):
        b = j % 2
        gth(j, b).wait()
        if j + 1 < JMAX:
          if j >= 1:
            sct(j - 1, (j - 1) % 2).wait()
          gth(j + 1, (j + 1) % 2).start()
        sct(j, b).start(add=True)
      sct(JMAX - 2, JMAX % 2).wait()
      sct(JMAX - 1, (JMAX - 1) % 2).wait()

    def wstart(w, b):
      base = s * 784 + w * 16
      pltpu.make_async_copy(src_hbm.at[pl.ds(base, 16)], srcw.at[b], wsem
                            ).start()
      pltpu.make_async_copy(dst_hbm.at[pl.ds(base, 16)], dstw.at[b], wsem
                            ).start()

    def wwait(w, b):
      base = s * 784 + w * 16
      pltpu.make_async_copy(src_hbm.at[pl.ds(base, 16)], srcw.at[b], wsem
                            ).wait()
      pltpu.make_async_copy(dst_hbm.at[pl.ds(base, 16)], dstw.at[b], wsem
                            ).wait()

    def rounds(rr, _):
      ck = rr * NSC + cc
      lo = ck * C
      limit = jnp.minimum(lo + C, N) - lo
      st = jnp.minimum(s * Q, limit - Q)
      pltpu.sync_copy(t_hbm.at[pl.ds(lo + st, Q)], acc_sp.at[pl.ds(st, Q)])
      plsc.subcore_barrier()

      wstart(0, 0)

      def win(w, cur):
        b = w % 2
        wwait(w, b)
        @pl.when(w + 1 < 49)
        def _():
          wstart(w + 1, (w + 1) % 2)
        base = s * 784 + w * 16
        nrows = jnp.clip(EROWS - base, 0, 16)

        def row(r, cur):
          for jj in range(8):
            dv = dstw[b, r, pl.ds(jj * 16, 16)]
            sv = srcw[b, r, pl.ds(jj * 16, 16)]
            m = jnp.logical_and(dv >= lo, dv < lo + C)
            plsc.store_compressed(dapp.at[pl.ds(cur, 16)], dv - lo, mask=m)
            plsc.store_compressed(sapp.at[pl.ds(cur, 16)], sv, mask=m)
            cur = cur + plsc.all_reduce_population_count(m)[0]

          def do_fire(cur):
            fire()
            for jj in range(8):
              sapp[pl.ds(jj * 16, 16)] = sapp[pl.ds(B_FIRE + jj * 16, 16)]
              dapp[pl.ds(jj * 16, 16)] = dapp[pl.ds(B_FIRE + jj * 16, 16)]
            return cur - B_FIRE
          return lax.cond(cur >= B_FIRE, do_fire, lambda cur: cur, cur)

        return lax.fori_loop(0, nrows, row, cur)

      cur = lax.fori_loop(0, 49, win, jnp.int32(0))

      def pad(jj, _):
        idx = jj * 16 + lane
        keep = idx < cur
        dsrc = jnp.remainder(idx * 797 + s * 4099, N)
        sapp[pl.ds(jj * 16, 16)] = jnp.where(keep, sapp[pl.ds(jj * 16, 16)],
                                             dsrc)
        dapp[pl.ds(jj * 16, 16)] = jnp.where(keep, dapp[pl.ds(jj * 16, 16)],
                                             C + lane)
        return 0
      lax.fori_loop(0, B_CAP // 16, pad, 0)
      fire()

      plsc.subcore_barrier()
      pltpu.sync_copy(acc_sp.at[pl.ds(st, Q)],
                      out_hbm.at[pl.ds(lo + st, Q)])
      plsc.subcore_barrier()
      return 0

    lax.fori_loop(0, rounds_n, rounds, 0)

  return k


_agg64 = _make_agg(64, 16800, 3)
_agg128 = _make_agg(128, 10000, 5)


BLK = 2000
NBLK = N // BLK


def _dinv_kernel(deg_partT):
  def body(deg_ref, out_ref):
    d = deg_ref[:, 0:1] + deg_ref[:, 1:2] + 1.0
    out_ref[...] = lax.rsqrt(d)
  return pl.pallas_call(
      body,
      grid=(NBLK,),
      in_specs=[pl.BlockSpec((BLK, NSC), lambda i: (i, 0))],
      out_specs=pl.BlockSpec((BLK, 1), lambda i: (i, 0)),
      out_shape=jax.ShapeDtypeStruct((N, 1), jnp.float32),
  )(deg_partT)


def _mm(h, Wm, dinv2, bias2, relu_pre):
  Fin, Fout = Wm.shape
  def body(h_ref, w_ref, dinv_ref, b_ref, out_ref):
    hb = h_ref[...]
    dv = dinv_ref[...]
    if relu_pre:
      hb = jnp.maximum(hb * dv + b_ref[...], 0.0)
    t = jnp.dot(hb, w_ref[...], preferred_element_type=jnp.float32)
    out_ref[...] = t * dv
  return pl.pallas_call(
      body,
      grid=(NBLK,),
      in_specs=[
          pl.BlockSpec((BLK, Fin), lambda i: (i, 0)),
          pl.BlockSpec((Fin, Fout), lambda i: (0, 0)),
          pl.BlockSpec((BLK, 1), lambda i: (i, 0)),
          pl.BlockSpec((1, Fin), lambda i: (0, 0)),
      ],
      out_specs=pl.BlockSpec((BLK, Fout), lambda i: (i, 0)),
      out_shape=jax.ShapeDtypeStruct((N, Fout), jnp.float32),
  )(h, Wm, dinv2, bias2)


def _pool_head(agg3, dinv2, b32, batch2d, Wl, bl2):
  def body(a_ref, dinv_ref, b3_ref, bt_ref, wl_ref, bl_ref, out_ref,
           s_acc, cnt_acc):
    i = pl.program_id(0)
    @pl.when(i == 0)
    def _():
      s_acc[...] = jnp.zeros_like(s_acc)
      cnt_acc[...] = jnp.zeros_like(cnt_acc)
    h = a_ref[...] * dinv_ref[...] + b3_ref[...]
    gids = lax.broadcasted_iota(jnp.int32, (BLK, G), 1)
    oh = (gids == bt_ref[...]).astype(jnp.float32)
    dnum = (((0,), (0,)), ((), ()))
    s_acc[...] += lax.dot_general(oh, h, dimension_numbers=dnum,
                                  preferred_element_type=jnp.float32)
    cnt_acc[...] += lax.dot_general(oh, jnp.ones((BLK, 1), jnp.float32),
                                    dimension_numbers=dnum,
                                    preferred_element_type=jnp.float32)
    @pl.when(i == NBLK - 1)
    def _():
      pooled = s_acc[...] / jnp.maximum(cnt_acc[...], 1.0)
      logits = jnp.dot(pooled, wl_ref[...],
                       preferred_element_type=jnp.float32) + bl_ref[...]
      mx = jnp.max(logits, axis=-1, keepdims=True)
      lse = mx + jnp.log(jnp.sum(jnp.exp(logits - mx), axis=-1, keepdims=True))
      out_ref[...] = logits - lse
  return pl.pallas_call(
      body,
      grid=(NBLK,),
      in_specs=[
          pl.BlockSpec((BLK, 128), lambda i: (i, 0)),
          pl.BlockSpec((BLK, 1), lambda i: (i, 0)),
          pl.BlockSpec((1, 128), lambda i: (0, 0)),
          pl.BlockSpec((BLK, 1), lambda i: (i, 0)),
          pl.BlockSpec((128, 14), lambda i: (0, 0)),
          pl.BlockSpec((1, 14), lambda i: (0, 0)),
      ],
      out_specs=pl.BlockSpec((G, 14), lambda i: (0, 0)),
      out_shape=jax.ShapeDtypeStruct((G, 14), jnp.float32),
      scratch_shapes=[
          pltpu.VMEM((G, 128), jnp.float32),
          pltpu.VMEM((G, 1), jnp.float32),
      ],
  )(agg3, dinv2, b32, batch2d, Wl, bl2)


def kernel(x, edge_index, batch, W1, b1, W2, b2, W3, b3, Wl, bl):
  pad = PADROWS * 128 - E
  src2d = jnp.concatenate(
      [edge_index[0], jnp.zeros((pad,), jnp.int32)]).reshape(PADROWS, 128)
  dst2d = jnp.concatenate(
      [edge_index[1], jnp.zeros((pad,), jnp.int32)]).reshape(PADROWS, 128)

  deg_part = _deg_kernel(dst2d)
  dinv2 = _dinv_kernel(deg_part.T)

  zero20 = jnp.zeros((1, 20), jnp.float32)
  t1 = _mm(x, W1, dinv2, zero20, relu_pre=False)
  a1 = _agg64(t1, src2d, dst2d)
  t2 = _mm(a1, W2, dinv2, b1.reshape(1, -1), relu_pre=True)
  a2 = _agg64(t2, src2d, dst2d)
  t3 = _mm(a2, W3, dinv2, b2.reshape(1, -1), relu_pre=True)
  a3 = _agg128(t3, src2d, dst2d)

  return _pool_head(a3, dinv2, b3.reshape(1, -1), batch.reshape(N, 1), Wl,
                    bl.reshape(1, -1))


# speedup vs baseline: 19.9744x; 1.1127x over previous
import functools

import jax
import jax.numpy as jnp
from jax import lax
from jax.experimental import pallas as pl
from jax.experimental.pallas import tpu as pltpu
from jax.experimental.pallas import tpu_sc as plsc

N = 100000
E = 1600000
G = 512
EROWS = E // 128
PADROWS = 12800
NSC = 2
NTILE = 16
C = 11200
NCHUNK = 9
ROUNDS = 5
B_FIRE = 1024
B_CAP = B_FIRE + 128
JMAX = B_FIRE // 128

_mesh = functools.partial(
    plsc.VectorSubcoreMesh, core_axis_name="c", subcore_axis_name="s")


def _zero_vmem(ref, n):
  z = jnp.zeros((16,), ref.dtype)
  def body(j, _):
    off = jnp.minimum(j * 16, n - 16)
    ref[pl.ds(off, 16)] = z
    return 0
  lax.fori_loop(0, (n + 15) // 16, body, 0)


def _deg_kernel(dst2d):
  @functools.partial(
      pl.kernel,
      out_type=jax.ShapeDtypeStruct((NSC, N), jnp.float32),
      mesh=_mesh(),
      scratch_types=[
          pltpu.VMEM((16, 128), jnp.int32),
          pltpu.VMEM((128,), jnp.float32),
          pltpu.VMEM((6256,), jnp.float32),
          pltpu.VMEM_SHARED((N,), jnp.float32),
      ],
      compiler_params=pltpu.CompilerParams(needs_layout_passes=False),
  )
  def k(dst_hbm, out_hbm, dstw, ones_v, zbuf, deg_sp):
    c = lax.axis_index("c")
    s = lax.axis_index("s")
    w32 = s * NSC + c

    one = jnp.ones((16,), jnp.float32)
    for j in range(8):
      ones_v[pl.ds(j * 16, 16)] = one
    _zero_vmem(zbuf, 6256)
    zst = jnp.minimum(s * 6256, N - 6256)
    pltpu.sync_copy(zbuf, deg_sp.at[pl.ds(zst, 6256)])
    plsc.subcore_barrier()

    def win(w, _):
      base = w32 * 400 + w * 16
      pltpu.sync_copy(dst_hbm.at[pl.ds(base, 16)], dstw)
      def row(r, _):
        @pl.when(base + r < EROWS)
        def _():
          pltpu.sync_copy(ones_v, deg_sp.at[dstw.at[r]], add=True)
        return 0
      lax.fori_loop(0, 16, row, 0)
      return 0
    lax.fori_loop(0, 25, win, 0)

    plsc.subcore_barrier()
    @pl.when(s == 0)
    def _():
      pltpu.sync_copy(deg_sp, out_hbm.at[c])

  return k(dst2d)


CB = 10000
NCH = 10
RND = 5
CAPSEG = 50176
FLUSH = 2048
FCAP = FLUSH + 128
QE = 632


def _bucket_kernel(src2d, dst2d):
  @functools.partial(
      pl.kernel,
      out_type=(jax.ShapeDtypeStruct((32 * NCH * CAPSEG,), jnp.int32),
                jax.ShapeDtypeStruct((32, NCH, 16), jnp.int32)),
      mesh=_mesh(),
      scratch_types=[
          pltpu.VMEM((2, 16, 128), jnp.int32),
          pltpu.VMEM((2, 16, 128), jnp.int32),
          pltpu.VMEM((NCH, FCAP), jnp.int32),
          pltpu.VMEM((NCH, 16), jnp.int32),
          pltpu.SemaphoreType.DMA,
      ],
      compiler_params=pltpu.CompilerParams(needs_layout_passes=False,
                                           use_tc_tiling_on_sc=False),
  )
  def k(src_hbm, dst_hbm, pk_hbm, cnts_hbm, srcw, dstw, bufs, cbuf, wsem):
    cc = lax.axis_index("c")
    s = lax.axis_index("s")
    st = s * NSC + cc
    lane = lax.iota(jnp.int32, 16)

    def wstart(w, b):
      base = st * 400 + w * 16
      pltpu.make_async_copy(src_hbm.at[pl.ds(base, 16)], srcw.at[b], wsem
                            ).start()
      pltpu.make_async_copy(dst_hbm.at[pl.ds(base, 16)], dstw.at[b], wsem
                            ).start()

    def wwait(w, b):
      base = st * 400 + w * 16
      pltpu.make_async_copy(src_hbm.at[pl.ds(base, 16)], srcw.at[b], wsem
                            ).wait()
      pltpu.make_async_copy(dst_hbm.at[pl.ds(base, 16)], dstw.at[b], wsem
                            ).wait()

    def flush_k(k_, cur, fl):
      base_e = (st * NCH + k_) * CAPSEG + fl * FLUSH
      pltpu.sync_copy(bufs.at[k_, pl.ds(0, FLUSH)],
                      pk_hbm.at[pl.ds(base_e, FLUSH)])
      for jj in range(8):
        bufs[k_, pl.ds(jj * 16, 16)] = bufs[k_, pl.ds(FLUSH + jj * 16, 16)]
      return cur - FLUSH, fl + 1

    wstart(0, 0)

    def win(w, carry):
      cur, fl = carry
      b = w % 2
      wwait(w, b)
      @pl.when(w + 1 < 25)
      def _():
        wstart(w + 1, (w + 1) % 2)
      base = st * 400 + w * 16
      nrows = jnp.clip(EROWS - base, 0, 16)

      def row(r, carry):
        cur, fl = carry
        for jj in range(8):
          dv = dstw[b, r, pl.ds(jj * 16, 16)]
          sv = srcw[b, r, pl.ds(jj * 16, 16)]
          ckv = dv // CB
          pk = sv | ((dv - ckv * CB) << 17)
          ncur = []
          for k_ in range(NCH):
            m = ckv == k_
            plsc.store_compressed(bufs.at[k_, pl.ds(cur[k_], 16)], pk, mask=m)
            ncur.append(cur[k_] + plsc.all_reduce_population_count(m)[0])
          cur = tuple(ncur)

        anyf = cur[0] >= FLUSH
        for k_ in range(1, NCH):
          anyf = jnp.logical_or(anyf, cur[k_] >= FLUSH)

        def do_flush(carry):
          cur, fl = carry
          ncur, nfl = list(cur), list(fl)
          for k_ in range(NCH):
            def yes(_c=k_):
              return flush_k(_c, cur[_c], fl[_c])
            def no(_c=k_):
              return cur[_c], fl[_c]
            ncur[k_], nfl[k_] = lax.cond(cur[k_] >= FLUSH, yes, no)
          return tuple(ncur), tuple(nfl)
        return lax.cond(anyf, do_flush, lambda c: c, (cur, fl))

      return lax.fori_loop(0, nrows, row, (cur, fl))

    zero = jnp.int32(0)
    cur, fl = lax.fori_loop(0, 25, win, ((zero,) * NCH, (zero,) * NCH))

    for k_ in range(NCH):
      def pad(jj, _):
        idx = jj * 16 + lane
        keep = idx < cur[k_]
        dsrc = jnp.remainder(idx * 797 + st * 4099, N)
        dummy = dsrc | ((CB + lane) << 17)
        bufs[k_, pl.ds(jj * 16, 16)] = jnp.where(
            keep, bufs[k_, pl.ds(jj * 16, 16)], dummy)
        return 0
      lax.fori_loop(0, FLUSH // 16, pad, 0)
      base_e = (st * NCH + k_) * CAPSEG + fl[k_] * FLUSH
      pltpu.sync_copy(bufs.at[k_, pl.ds(0, FLUSH)],
                      pk_hbm.at[pl.ds(base_e, FLUSH)])
      cbuf[k_, pl.ds(0, 16)] = jnp.broadcast_to(fl[k_] * FLUSH + cur[k_], (16,))
    pltpu.sync_copy(cbuf, cnts_hbm.at[st])

  return k(src2d, dst2d)


def _make_agg(F):
  @functools.partial(
      pl.kernel,
      out_type=jax.ShapeDtypeStruct((N, F), jnp.float32),
      mesh=_mesh(),
      scratch_types=[
          pltpu.VMEM((1024,), jnp.int32),
          pltpu.VMEM((1024,), jnp.int32),
          pltpu.VMEM((8, 128), jnp.int32),
          pltpu.VMEM((2, 128, F), jnp.float32),
          pltpu.VMEM((NCH, 16), jnp.int32),
          pltpu.VMEM_SHARED((CB + 16, F), jnp.float32),
          pltpu.SemaphoreType.DMA,
          pltpu.SemaphoreType.DMA,
      ],
      compiler_params=pltpu.CompilerParams(needs_layout_passes=False,
                                           use_tc_tiling_on_sc=False),
  )
  def k(t_hbm, pk_hbm, cnts_hbm, out_hbm,
        pkw, sfire, dfire, rows, cbuf, acc_sp, gsem, ssem):
    cc = lax.axis_index("c")
    s = lax.axis_index("s")

    def fire():
      def gth(j, b):
        return pltpu.make_async_copy(
            t_hbm.at[sfire.at[pl.ds(j * 128, 128)]], rows.at[b], gsem)
      def sct(j, b):
        return pltpu.make_async_copy(rows.at[b], acc_sp.at[dfire.at[j]], ssem)
      gth(0, 0).start()
      for j in range(JMAX):
        b = j % 2
        gth(j, b).wait()
        if j + 1 < JMAX:
          if j >= 1:
            sct(j - 1, (j - 1) % 2).wait()
          gth(j + 1, (j + 1) % 2).start()
        sct(j, b).start(add=True)
      sct(JMAX - 2, JMAX % 2).wait()
      sct(JMAX - 1, (JMAX - 1) % 2).wait()

    def rounds(rr, _):
      ck = rr * NSC + cc
      lo = ck * CB
      st = jnp.minimum(s * QE, CB - QE)
      pltpu.sync_copy(t_hbm.at[pl.ds(lo + st, QE)], acc_sp.at[pl.ds(st, QE)])
      plsc.subcore_barrier()

      for half in range(2):
        bt = s * 2 + half
        pltpu.sync_copy(cnts_hbm.at[bt], cbuf)
        cnt = cbuf[ck, pl.ds(0, 16)][0]
        base_e = (bt * NCH + ck) * CAPSEG

        def fire_w(w, _):
          pltpu.sync_copy(pk_hbm.at[pl.ds(base_e + w * 1024, 1024)], pkw)
          for jj in range(64):
            pkv = pkw[pl.ds(jj * 16, 16)]
            sfire[pl.ds(jj * 16, 16)] = pkv & 0x1FFFF
            dfire[jj // 8, pl.ds((jj % 8) * 16, 16)] = (
                lax.shift_right_logical(pkv, 17))
          fire()
          return 0
        lax.fori_loop(0, (cnt + 1023) // 1024, fire_w, 0)

      plsc.subcore_barrier()
      pltpu.sync_copy(acc_sp.at[pl.ds(st, QE)],
                      out_hbm.at[pl.ds(lo + st, QE)])
      plsc.subcore_barrier()
      return 0

    lax.fori_loop(0, RND, rounds, 0)

  return k


_agg64 = _make_agg(64)
_agg128 = _make_agg(128)


BLK = 2000
NBLK = N // BLK


def _dinv_kernel(deg_partT):
  def body(deg_ref, out_ref):
    d = deg_ref[:, 0:1] + deg_ref[:, 1:2] + 1.0
    out_ref[...] = lax.rsqrt(d)
  return pl.pallas_call(
      body,
      grid=(NBLK,),
      in_specs=[pl.BlockSpec((BLK, NSC), lambda i: (i, 0))],
      out_specs=pl.BlockSpec((BLK, 1), lambda i: (i, 0)),
      out_shape=jax.ShapeDtypeStruct((N, 1), jnp.float32),
  )(deg_partT)


def _mm(h, Wm, dinv2, bias2, relu_pre):
  Fin, Fout = Wm.shape
  def body(h_ref, w_ref, dinv_ref, b_ref, out_ref):
    hb = h_ref[...]
    dv = dinv_ref[...]
    if relu_pre:
      hb = jnp.maximum(hb * dv + b_ref[...], 0.0)
    t = jnp.dot(hb, w_ref[...], preferred_element_type=jnp.float32)
    out_ref[...] = t * dv
  return pl.pallas_call(
      body,
      grid=(NBLK,),
      in_specs=[
          pl.BlockSpec((BLK, Fin), lambda i: (i, 0)),
          pl.BlockSpec((Fin, Fout), lambda i: (0, 0)),
          pl.BlockSpec((BLK, 1), lambda i: (i, 0)),
          pl.BlockSpec((1, Fin), lambda i: (0, 0)),
      ],
      out_specs=pl.BlockSpec((BLK, Fout), lambda i: (i, 0)),
      out_shape=jax.ShapeDtypeStruct((N, Fout), jnp.float32),
  )(h, Wm, dinv2, bias2)


def _pool_head(agg3, dinv2, b32, batch2d, Wl, bl2):
  def body(a_ref, dinv_ref, b3_ref, bt_ref, wl_ref, bl_ref, out_ref,
           s_acc, cnt_acc):
    i = pl.program_id(0)
    @pl.when(i == 0)
    def _():
      s_acc[...] = jnp.zeros_like(s_acc)
      cnt_acc[...] = jnp.zeros_like(cnt_acc)
    h = a_ref[...] * dinv_ref[...] + b3_ref[...]
    gids = lax.broadcasted_iota(jnp.int32, (BLK, G), 1)
    oh = (gids == bt_ref[...]).astype(jnp.float32)
    dnum = (((0,), (0,)), ((), ()))
    s_acc[...] += lax.dot_general(oh, h, dimension_numbers=dnum,
                                  preferred_element_type=jnp.float32)
    cnt_acc[...] += lax.dot_general(oh, jnp.ones((BLK, 1), jnp.float32),
                                    dimension_numbers=dnum,
                                    preferred_element_type=jnp.float32)
    @pl.when(i == NBLK - 1)
    def _():
      pooled = s_acc[...] / jnp.maximum(cnt_acc[...], 1.0)
      logits = jnp.dot(pooled, wl_ref[...],
                       preferred_element_type=jnp.float32) + bl_ref[...]
      mx = jnp.max(logits, axis=-1, keepdims=True)
      lse = mx + jnp.log(jnp.sum(jnp.exp(logits - mx), axis=-1, keepdims=True))
      out_ref[...] = logits - lse
  return pl.pallas_call(
      body,
      grid=(NBLK,),
      in_specs=[
          pl.BlockSpec((BLK, 128), lambda i: (i, 0)),
          pl.BlockSpec((BLK, 1), lambda i: (i, 0)),
          pl.BlockSpec((1, 128), lambda i: (0, 0)),
          pl.BlockSpec((BLK, 1), lambda i: (i, 0)),
          pl.BlockSpec((128, 14), lambda i: (0, 0)),
          pl.BlockSpec((1, 14), lambda i: (0, 0)),
      ],
      out_specs=pl.BlockSpec((G, 14), lambda i: (0, 0)),
      out_shape=jax.ShapeDtypeStruct((G, 14), jnp.float32),
      scratch_shapes=[
          pltpu.VMEM((G, 128), jnp.float32),
          pltpu.VMEM((G, 1), jnp.float32),
      ],
  )(agg3, dinv2, b32, batch2d, Wl, bl2)


def kernel(x, edge_index, batch, W1, b1, W2, b2, W3, b3, Wl, bl):
  pad = PADROWS * 128 - E
  src2d = jnp.concatenate(
      [edge_index[0], jnp.zeros((pad,), jnp.int32)]).reshape(PADROWS, 128)
  dst2d = jnp.concatenate(
      [edge_index[1], jnp.zeros((pad,), jnp.int32)]).reshape(PADROWS, 128)

  deg_part = _deg_kernel(dst2d)
  pk, cnts = _bucket_kernel(src2d, dst2d)
  dinv2 = _dinv_kernel(deg_part.T)

  zero20 = jnp.zeros((1, 20), jnp.float32)
  t1 = _mm(x, W1, dinv2, zero20, relu_pre=False)
  a1 = _agg64(t1, pk, cnts)
  t2 = _mm(a1, W2, dinv2, b1.reshape(1, -1), relu_pre=True)
  a2 = _agg64(t2, pk, cnts)
  t3 = _mm(a2, W3, dinv2, b2.reshape(1, -1), relu_pre=True)
  a3 = _agg128(t3, pk, cnts)

  return _pool_head(a3, dinv2, b3.reshape(1, -1), batch.reshape(N, 1), Wl,
                    bl.reshape(1, -1))
